# baseline (speedup 1.0000x reference)
import jax
import jax.numpy as jnp
from jax.experimental import pallas as pl
from jax.experimental.pallas import tpu as pltpu

_EMBED = 384
_NQ = 32
_K = 8
_CORPUS = 100000
_BLK = 16384
_NB = (_CORPUS + _BLK - 1) // _BLK


def _retrieve_kernel(q_ref, k_ref, outs_ref, outi_ref, s_ref):
    i = pl.program_id(0)

    @pl.when(i == 0)
    def _init():
        outs_ref[...] = jnp.full((_NQ, _K), -jnp.inf, jnp.float32)
        outi_ref[...] = jnp.zeros((_NQ, _K), jnp.int32)

    q = q_ref[...]
    kb = k_ref[...]
    norm = jnp.sqrt(jnp.sum(kb * kb, axis=1, keepdims=True))
    kn = kb * (1.0 / jnp.maximum(norm, 1e-12))
    s = jax.lax.dot_general(
        q, kn.astype(jnp.bfloat16),
        (((1,), (1,)), ((), ())),
        preferred_element_type=jnp.float32,
    )
    lane = jax.lax.broadcasted_iota(jnp.int32, (_NQ, _BLK), 1)

    @pl.when(i < _NB - 1)
    def _store():
        s_ref[...] = s

    @pl.when(i == _NB - 1)
    def _store_masked():
        gidx = lane + i * _BLK
        s_ref[...] = jnp.where(gidx < _CORPUS, s, -jnp.inf)

    pos = jax.lax.broadcasted_iota(jnp.int32, (_NQ, _K), 1)

    def extract(_, done):
        def run():
            sv = s_ref[...]
            m = jnp.max(sv, axis=1, keepdims=True)
            am = jnp.min(
                jnp.where(sv == m, lane, jnp.int32(2 ** 30)),
                axis=1, keepdims=True,
            )
            bs = outs_ref[...]
            bi = outi_ref[...]
            improved = m[:, 0] > bs[:, _K - 1]
            r = jnp.sum((bs >= m).astype(jnp.int32), axis=1, keepdims=True)
            shs = jnp.concatenate([bs[:, :1], bs[:, : _K - 1]], axis=1)
            shi = jnp.concatenate([bi[:, :1], bi[:, : _K - 1]], axis=1)
            gm = am + i * _BLK
            outs_ref[...] = jnp.where(pos < r, bs, jnp.where(pos == r, m, shs))
            outi_ref[...] = jnp.where(pos < r, bi, jnp.where(pos == r, gm, shi))
            s_ref[...] = jnp.where(lane == am, -jnp.inf, sv)
            return jnp.logical_not(jnp.any(improved)).astype(jnp.int32)

        return jax.lax.cond(done != 0, lambda: jnp.int32(1), run)

    jax.lax.fori_loop(0, _K, extract, jnp.int32(0))


@jax.jit
def kernel(queries, keys):
    qn = queries / jnp.clip(
        jnp.linalg.norm(queries, axis=1, keepdims=True), 1e-12, None
    )
    qn = qn.astype(jnp.bfloat16)
    outs, outi = pl.pallas_call(
        _retrieve_kernel,
        grid=(_NB,),
        in_specs=[
            pl.BlockSpec((_NQ, _EMBED), lambda i: (0, 0)),
            pl.BlockSpec((_BLK, _EMBED), lambda i: (i, 0)),
        ],
        out_specs=[
            pl.BlockSpec((_NQ, _K), lambda i: (0, 0)),
            pl.BlockSpec((_NQ, _K), lambda i: (0, 0)),
        ],
        out_shape=[
            jax.ShapeDtypeStruct((_NQ, _K), jnp.float32),
            jax.ShapeDtypeStruct((_NQ, _K), jnp.int32),
        ],
        scratch_shapes=[pltpu.VMEM((_NQ, _BLK), jnp.float32)],
    )(qn, keys)
    return outs, outi


# speedup vs baseline: 1.1841x; 1.1841x over previous
import jax
import jax.numpy as jnp
from jax.experimental import pallas as pl
from jax.experimental.pallas import tpu as pltpu

_EMBED = 384
_NQ = 32
_K = 8
_CORPUS = 100000
_BLK = 8192
_NB = (_CORPUS + _BLK - 1) // _BLK


def _retrieve_kernel(q_ref, k_ref, outs_ref, outi_ref, s_ref):
    i = pl.program_id(0)

    @pl.when(i == 0)
    def _init():
        outs_ref[...] = jnp.full((_NQ, _K), -jnp.inf, jnp.float32)
        outi_ref[...] = jnp.zeros((_NQ, _K), jnp.int32)

    q = q_ref[...]
    kb = k_ref[...]
    nsq = jnp.sum(kb * kb, axis=1, keepdims=True)
    kn = kb * jax.lax.rsqrt(jnp.maximum(nsq, 1e-24))
    s = jax.lax.dot_general(
        q, kn.astype(jnp.bfloat16),
        (((1,), (1,)), ((), ())),
        preferred_element_type=jnp.float32,
    )
    lane = jax.lax.broadcasted_iota(jnp.int32, (_NQ, _BLK), 1)

    @pl.when(i < _NB - 1)
    def _store():
        s_ref[...] = s

    @pl.when(i == _NB - 1)
    def _store_masked():
        gidx = lane + i * _BLK
        s_ref[...] = jnp.where(gidx < _CORPUS, s, -jnp.inf)

    pos = jax.lax.broadcasted_iota(jnp.int32, (_NQ, _K), 1)

    def extract(_, done):
        def run():
            sv = s_ref[...]
            m = jnp.max(sv, axis=1, keepdims=True)
            am = jnp.min(
                jnp.where(sv == m, lane, jnp.int32(2 ** 30)),
                axis=1, keepdims=True,
            )
            bs = outs_ref[...]
            bi = outi_ref[...]
            improved = m[:, 0] > bs[:, _K - 1]
            r = jnp.sum((bs >= m).astype(jnp.int32), axis=1, keepdims=True)
            shs = jnp.concatenate([bs[:, :1], bs[:, : _K - 1]], axis=1)
            shi = jnp.concatenate([bi[:, :1], bi[:, : _K - 1]], axis=1)
            gm = am + i * _BLK
            outs_ref[...] = jnp.where(pos < r, bs, jnp.where(pos == r, m, shs))
            outi_ref[...] = jnp.where(pos < r, bi, jnp.where(pos == r, gm, shi))
            s_ref[...] = jnp.where(lane == am, -jnp.inf, sv)
            return jnp.logical_not(jnp.any(improved)).astype(jnp.int32)

        return jax.lax.cond(done != 0, lambda: jnp.int32(1), run)

    jax.lax.fori_loop(0, _K, extract, jnp.int32(0))


@jax.jit
def kernel(queries, keys):
    qn = queries / jnp.clip(
        jnp.linalg.norm(queries, axis=1, keepdims=True), 1e-12, None
    )
    qn = qn.astype(jnp.bfloat16)
    outs, outi = pl.pallas_call(
        _retrieve_kernel,
        grid=(_NB,),
        in_specs=[
            pl.BlockSpec((_NQ, _EMBED), lambda i: (0, 0)),
            pl.BlockSpec((_BLK, _EMBED), lambda i: (i, 0)),
        ],
        out_specs=[
            pl.BlockSpec((_NQ, _K), lambda i: (0, 0)),
            pl.BlockSpec((_NQ, _K), lambda i: (0, 0)),
        ],
        out_shape=[
            jax.ShapeDtypeStruct((_NQ, _K), jnp.float32),
            jax.ShapeDtypeStruct((_NQ, _K), jnp.int32),
        ],
        scratch_shapes=[pltpu.VMEM((_NQ, _BLK), jnp.float32)],
    )(qn, keys)
    return outs, outi


# speedup vs baseline: 1.2664x; 1.0695x over previous
import jax
import jax.numpy as jnp
from jax.experimental import pallas as pl
from jax.experimental.pallas import tpu as pltpu

_EMBED = 384
_NQ = 32
_K = 8
_CORPUS = 100000
_BLK = 8192
_NB = (_CORPUS + _BLK - 1) // _BLK


def _retrieve_kernel(q_ref, k_ref, outs_ref, outi_ref, s_ref):
    i = pl.program_id(0)

    @pl.when(i == 0)
    def _init():
        outs_ref[...] = jnp.full((_NQ, _K), -jnp.inf, jnp.float32)
        outi_ref[...] = jnp.zeros((_NQ, _K), jnp.int32)

    q = q_ref[...]
    kb = k_ref[...]
    nsq = jnp.sum(kb * kb, axis=1, keepdims=True)
    kn = kb * jax.lax.rsqrt(jnp.maximum(nsq, 1e-24))
    s = jax.lax.dot_general(
        q, kn.astype(jnp.bfloat16),
        (((1,), (1,)), ((), ())),
        preferred_element_type=jnp.float32,
    )
    lane = jax.lax.broadcasted_iota(jnp.int32, (_NQ, _BLK), 1)

    @pl.when(i < _NB - 1)
    def _store():
        s_ref[...] = s

    @pl.when(i == _NB - 1)
    def _store_masked():
        gidx = lane + i * _BLK
        s_ref[...] = jnp.where(gidx < _CORPUS, s, -jnp.inf)

    pos = jax.lax.broadcasted_iota(jnp.int32, (_NQ, _K), 1)

    def extract(_, done):
        def run():
            sv = s_ref[...]
            m = jnp.max(sv, axis=1, keepdims=True)
            am = jnp.argmax(sv, axis=1).astype(jnp.int32)[:, None]
            bs = outs_ref[...]
            bi = outi_ref[...]
            improved = m[:, 0] > bs[:, _K - 1]
            r = jnp.sum((bs >= m).astype(jnp.int32), axis=1, keepdims=True)
            shs = jnp.concatenate([bs[:, :1], bs[:, : _K - 1]], axis=1)
            shi = jnp.concatenate([bi[:, :1], bi[:, : _K - 1]], axis=1)
            gm = am + i * _BLK
            outs_ref[...] = jnp.where(pos < r, bs, jnp.where(pos == r, m, shs))
            outi_ref[...] = jnp.where(pos < r, bi, jnp.where(pos == r, gm, shi))
            s_ref[...] = jnp.where(lane == am, -jnp.inf, sv)
            return jnp.logical_not(jnp.any(improved)).astype(jnp.int32)

        return jax.lax.cond(done != 0, lambda: jnp.int32(1), run)

    jax.lax.fori_loop(0, _K, extract, jnp.int32(0))


@jax.jit
def kernel(queries, keys):
    qn = queries / jnp.clip(
        jnp.linalg.norm(queries, axis=1, keepdims=True), 1e-12, None
    )
    qn = qn.astype(jnp.bfloat16)
    outs, outi = pl.pallas_call(
        _retrieve_kernel,
        grid=(_NB,),
        in_specs=[
            pl.BlockSpec((_NQ, _EMBED), lambda i: (0, 0)),
            pl.BlockSpec((_BLK, _EMBED), lambda i: (i, 0)),
        ],
        out_specs=[
            pl.BlockSpec((_NQ, _K), lambda i: (0, 0)),
            pl.BlockSpec((_NQ, _K), lambda i: (0, 0)),
        ],
        out_shape=[
            jax.ShapeDtypeStruct((_NQ, _K), jnp.float32),
            jax.ShapeDtypeStruct((_NQ, _K), jnp.int32),
        ],
        scratch_shapes=[pltpu.VMEM((_NQ, _BLK), jnp.float32)],
    )(qn, keys)
    return outs, outi
